# baseline (speedup 1.0000x reference)
import functools

import jax
import jax.numpy as jnp
from jax import lax
from jax.experimental import pallas as pl
from jax.experimental.pallas import tpu as pltpu
from jax.experimental.pallas import tpu_sc as plsc

D = 128
S = 10
L = 16
NW = 32
C = 32
R = C * S
CHUNKS = 50
PER_TILE = C * CHUNKS
BPAD = PER_TILE * NW
GATHER_SPLITS = ((0, 128), (128, 128), (256, 64))


def _sc_mean(features, idx_flat):
    mesh = plsc.VectorSubcoreMesh(core_axis_name="c", subcore_axis_name="s")

    @functools.partial(
        pl.kernel,
        mesh=mesh,
        out_type=jax.ShapeDtypeStruct((BPAD, D), jnp.float32),
        scratch_types=[
            pltpu.VMEM((2 * R,), jnp.int32),
            pltpu.VMEM((2 * R, D), jnp.float32),
            pltpu.VMEM((2 * C, D), jnp.float32),
            pltpu.SemaphoreType.DMA,
        ],
    )
    def k(feat_hbm, idx_hbm, out_hbm, idx_v, rows_v, out_v, gsem):
        wid = lax.axis_index("s") * 2 + lax.axis_index("c")
        tile_node0 = wid * PER_TILE
        tile_row0 = tile_node0 * S

        def pair_body(p, carry):
            c0 = 2 * p
            pltpu.sync_copy(idx_hbm.at[pl.ds(tile_row0 + c0 * R, 2 * R)],
                            idx_v)
            cps = []
            for half in (0, R):
                for g0, gn in GATHER_SPLITS:
                    cps.append(pltpu.async_copy(
                        feat_hbm.at[idx_v.at[pl.ds(half + g0, gn)]],
                        rows_v.at[pl.ds(half + g0, gn)],
                        gsem,
                    ))
            for cp in cps:
                cp.wait()

            def node_body(n, carry2):
                base = n * S
                for col in range(D // L):
                    acc = rows_v[base, pl.ds(col * L, L)]
                    for s_ in range(1, S):
                        acc = acc + rows_v[base + s_, pl.ds(col * L, L)]
                    out_v[n, pl.ds(col * L, L)] = acc * jnp.float32(0.1)
                return carry2

            lax.fori_loop(0, 2 * C, node_body, 0)
            pltpu.sync_copy(out_v,
                            out_hbm.at[pl.ds(tile_node0 + c0 * C, 2 * C)])
            return carry

        lax.fori_loop(0, CHUNKS // 2, pair_body, 0)

    return k(features, idx_flat)


def kernel(features, nodes, to_neighs):
    b = to_neighs.shape[0]
    idx = to_neighs.astype(jnp.int32).reshape(-1)
    idx = jnp.pad(idx, (0, BPAD * S - idx.shape[0]))
    out = _sc_mean(features, idx)
    return out[:b]


# speedup vs baseline: 1.0447x; 1.0447x over previous
import functools

import jax
import jax.numpy as jnp
from jax import lax
from jax.experimental import pallas as pl
from jax.experimental.pallas import tpu as pltpu
from jax.experimental.pallas import tpu_sc as plsc

D = 128
S = 10
L = 16
NW = 16
C = 32
R = C * S
CHUNKS = 100
PER_TILE = C * CHUNKS
BPAD = PER_TILE * NW
GATHER_SPLITS = ((0, 128), (128, 128), (256, 64))


def _sc_mean(features, idx_flat):
    mesh = plsc.VectorSubcoreMesh(core_axis_name="c", subcore_axis_name="s",
                                  num_cores=1)

    @functools.partial(
        pl.kernel,
        mesh=mesh,
        out_type=jax.ShapeDtypeStruct((BPAD, D), jnp.float32),
        scratch_types=[
            pltpu.VMEM((2 * R,), jnp.int32),
            pltpu.VMEM((2 * R, D), jnp.float32),
            pltpu.VMEM((2 * C, D), jnp.float32),
            pltpu.SemaphoreType.DMA,
            pltpu.SemaphoreType.DMA,
            pltpu.SemaphoreType.DMA,
        ],
    )
    def k(feat_hbm, idx_hbm, out_hbm, idx_v, rows_v, out_v, isem, gsem, osem):
        wid = lax.axis_index("s")
        tile_node0 = wid * PER_TILE
        tile_row0 = tile_node0 * S

        def i_start(c, boff):
            pltpu.async_copy(
                idx_hbm.at[pl.ds(tile_row0 + c * R, R)],
                idx_v.at[pl.ds(boff, R)], isem)

        def i_wait():
            pltpu.make_async_copy(
                idx_hbm.at[pl.ds(tile_row0, R)],
                idx_v.at[pl.ds(0, R)], isem).wait()

        def g_start(boff):
            for g0, gn in GATHER_SPLITS:
                pltpu.async_copy(
                    feat_hbm.at[idx_v.at[pl.ds(boff + g0, gn)]],
                    rows_v.at[pl.ds(boff + g0, gn)],
                    gsem,
                )

        def g_wait(boff):
            for g0, gn in GATHER_SPLITS:
                pltpu.make_async_copy(
                    feat_hbm.at[idx_v.at[pl.ds(boff + g0, gn)]],
                    rows_v.at[pl.ds(boff + g0, gn)],
                    gsem,
                ).wait()

        def o_start(c, ooff):
            pltpu.async_copy(
                out_v.at[pl.ds(ooff, C)],
                out_hbm.at[pl.ds(tile_node0 + c * C, C)], osem)

        def o_wait():
            pltpu.make_async_copy(
                out_v.at[pl.ds(0, C)],
                out_hbm.at[pl.ds(tile_node0, C)], osem).wait()

        pltpu.sync_copy(idx_hbm.at[pl.ds(tile_row0, R)],
                        idx_v.at[pl.ds(0, R)])
        g_start(0)
        i_start(1, R)

        def chunk_body(c, carry):
            par = lax.rem(c, 2)
            boff = par * R
            boff_n = R - boff
            ooff = par * C

            g_wait(boff)

            @pl.when(c + 2 < CHUNKS)
            def _():
                i_start(c + 2, boff)

            @pl.when(c + 1 < CHUNKS)
            def _():
                i_wait()
                g_start(boff_n)

            @pl.when(c >= 2)
            def _():
                o_wait()

            def node_body(n, carry2):
                base = boff + n * S
                for col in range(D // L):
                    acc = rows_v[base, pl.ds(col * L, L)]
                    for s_ in range(1, S):
                        acc = acc + rows_v[base + s_, pl.ds(col * L, L)]
                    out_v[ooff + n, pl.ds(col * L, L)] = acc * jnp.float32(0.1)
                return carry2

            lax.fori_loop(0, C, node_body, 0)
            o_start(c, ooff)
            return carry

        lax.fori_loop(0, CHUNKS, chunk_body, 0)
        o_wait()
        o_wait()

    return k(features, idx_flat)


def kernel(features, nodes, to_neighs):
    b = to_neighs.shape[0]
    idx = to_neighs.astype(jnp.int32).reshape(-1)
    idx = jnp.pad(idx, (0, BPAD * S - idx.shape[0]))
    out = _sc_mean(features, idx)
    return out[:b]


# speedup vs baseline: 1.5964x; 1.5281x over previous
import functools

import jax
import jax.numpy as jnp
from jax import lax
from jax.experimental import pallas as pl
from jax.experimental.pallas import tpu as pltpu
from jax.experimental.pallas import tpu_sc as plsc

D = 128
DW = D // 2
S = 10
L = 16
NW = 32
C = 32
R = C * S
CHUNKS = 49
PER_TILE = C * CHUNKS
BPAD = PER_TILE * NW
GATHER_SPLITS = ((0, 128), (128, 128), (256, 64))


def _sc_mean(features_pk, idx_flat):
    mesh = plsc.VectorSubcoreMesh(core_axis_name="c", subcore_axis_name="s")

    @functools.partial(
        pl.kernel,
        mesh=mesh,
        out_type=jax.ShapeDtypeStruct((BPAD, D), jnp.float32),
        compiler_params=pltpu.CompilerParams(needs_layout_passes=False, use_tc_tiling_on_sc=False),
        scratch_types=[
            pltpu.VMEM((R,), jnp.int32),
            pltpu.VMEM((R, DW), jnp.int32),
            pltpu.VMEM((C, D), jnp.float32),
            pltpu.SemaphoreType.DMA,
        ],
    )
    def k(feat_hbm, idx_hbm, out_hbm, idx_v, rows_v, out_v, sem):
        wid = lax.axis_index("s") * 2 + lax.axis_index("c")
        tile_node0 = wid * PER_TILE

        def chunk_body(ci, carry):
            node0 = tile_node0 + ci * C
            row0 = node0 * S
            pltpu.sync_copy(idx_hbm.at[pl.ds(row0, R)], idx_v)
            cps = []
            for g0, gn in GATHER_SPLITS:
                cps.append(pltpu.async_copy(
                    feat_hbm.at[idx_v.at[pl.ds(g0, gn)]],
                    rows_v.at[pl.ds(g0, gn)],
                    sem,
                ))
            for cp in cps:
                cp.wait()

            def node_body(n, carry2):
                base = n * S
                for g in range(DW // L):
                    acc_lo = None
                    acc_hi = None
                    for s_ in range(S):
                        w = rows_v[base + s_, pl.ds(g * L, L)]
                        lo = plsc.bitcast(w << 16, jnp.float32)
                        hi = plsc.bitcast(w & jnp.int32(-65536),
                                          jnp.float32)
                        acc_lo = lo if acc_lo is None else acc_lo + lo
                        acc_hi = hi if acc_hi is None else acc_hi + hi
                    out_v[n, pl.ds(g * 2 * L, L)] = acc_lo * jnp.float32(0.1)
                    out_v[n, pl.ds(g * 2 * L + L, L)] = (
                        acc_hi * jnp.float32(0.1))
                return carry2

            lax.fori_loop(0, C, node_body, 0)
            pltpu.sync_copy(out_v, out_hbm.at[pl.ds(node0, C)])
            return carry

        lax.fori_loop(0, CHUNKS, chunk_body, 0)

    return k(features_pk, idx_flat)


def kernel(features, nodes, to_neighs):
    b = to_neighs.shape[0]
    features_pk = jax.lax.bitcast_convert_type(
        features.astype(jnp.bfloat16).reshape(features.shape[0], DW, 2),
        jnp.int32)
    idx = to_neighs.astype(jnp.int32).reshape(-1)
    idx = jnp.pad(idx, (0, BPAD * S - idx.shape[0]))
    out_dei = _sc_mean(features_pk, idx)
    out = (out_dei.reshape(BPAD, D // 32, 2, L)
           .transpose(0, 1, 3, 2)
           .reshape(BPAD, D))
    return out[:b]


# speedup vs baseline: 2.1282x; 1.3332x over previous
import functools

import jax
import jax.numpy as jnp
from jax import lax
from jax.experimental import pallas as pl
from jax.experimental.pallas import tpu as pltpu
from jax.experimental.pallas import tpu_sc as plsc

D = 128
DW = D // 2
S = 10
L = 16
NW = 32
C = 32
R = C * S
CHUNKS = 49
PER_TILE = C * CHUNKS
GATHER_SPLITS = ((0, 128), (128, 128), (256, 64))


def _sc_mean(features_pk, idx_flat, batch):
    mesh = plsc.VectorSubcoreMesh(core_axis_name="c", subcore_axis_name="s")

    @functools.partial(
        pl.kernel,
        mesh=mesh,
        out_type=jax.ShapeDtypeStruct((batch, D), jnp.float32),
        compiler_params=pltpu.CompilerParams(needs_layout_passes=False,
                                             use_tc_tiling_on_sc=False),
        scratch_types=[
            pltpu.VMEM((R,), jnp.int32),
            pltpu.VMEM((R, DW), jnp.int32),
            pltpu.VMEM((C, D), jnp.float32),
            pltpu.SemaphoreType.DMA,
        ],
    )
    def k(feat_hbm, idx_hbm, out_hbm, idx_v, rows_v, out_v, sem):
        wid = lax.axis_index("s") * 2 + lax.axis_index("c")
        tile_node0 = jnp.minimum(wid * PER_TILE, batch - PER_TILE)
        iota2 = lax.iota(jnp.int32, L) * 2
        cols_lo = [iota2 + (2 * L) * g for g in range(DW // L)]
        cols_hi = [col + 1 for col in cols_lo]

        def chunk_body(ci, carry):
            node0 = tile_node0 + ci * C
            row0 = node0 * S
            pltpu.sync_copy(idx_hbm.at[pl.ds(row0, R)], idx_v)
            cps = []
            for g0, gn in GATHER_SPLITS:
                cps.append(pltpu.async_copy(
                    feat_hbm.at[idx_v.at[pl.ds(g0, gn)]],
                    rows_v.at[pl.ds(g0, gn)],
                    sem,
                ))
            for cp in cps:
                cp.wait()

            def node_body(n, carry2):
                base = n * S
                rown = jnp.full((L,), n, dtype=jnp.int32)
                for g in range(DW // L):
                    acc_lo = None
                    acc_hi = None
                    for s_ in range(S):
                        w = rows_v[base + s_, pl.ds(g * L, L)]
                        lo = plsc.bitcast(w << 16, jnp.float32)
                        hi = plsc.bitcast(w & jnp.int32(-65536),
                                          jnp.float32)
                        acc_lo = lo if acc_lo is None else acc_lo + lo
                        acc_hi = hi if acc_hi is None else acc_hi + hi
                    plsc.store_scatter(out_v, [rown, cols_lo[g]],
                                       acc_lo * jnp.float32(0.1))
                    plsc.store_scatter(out_v, [rown, cols_hi[g]],
                                       acc_hi * jnp.float32(0.1))
                return carry2

            lax.fori_loop(0, C, node_body, 0)
            pltpu.sync_copy(out_v, out_hbm.at[pl.ds(node0, C)])
            return carry

        lax.fori_loop(0, CHUNKS, chunk_body, 0)

    return k(features_pk, idx_flat)


def kernel(features, nodes, to_neighs):
    b = to_neighs.shape[0]
    features_pk = jax.lax.bitcast_convert_type(
        features.astype(jnp.bfloat16).reshape(features.shape[0], DW, 2),
        jnp.int32)
    idx = to_neighs.astype(jnp.int32).reshape(-1)
    return _sc_mean(features_pk, idx, b)


# speedup vs baseline: 3.4544x; 1.6231x over previous
import functools

import jax
import jax.numpy as jnp
from jax import lax
from jax.experimental import pallas as pl
from jax.experimental.pallas import tpu as pltpu
from jax.experimental.pallas import tpu_sc as plsc

D = 128
DW = D // 2
S = 10
L = 16
NW = 32
C = 32
R = C * S
CHUNKS = 49
PER_TILE = C * CHUNKS
GATHER_SPLITS = ((0, 128), (128, 128), (256, 64))


def _sc_mean(features_pk, idx_flat, batch):
    mesh = plsc.VectorSubcoreMesh(core_axis_name="c", subcore_axis_name="s")

    @functools.partial(
        pl.kernel,
        mesh=mesh,
        out_type=jax.ShapeDtypeStruct((batch, D), jnp.float32),
        compiler_params=pltpu.CompilerParams(needs_layout_passes=False,
                                             use_tc_tiling_on_sc=False),
        scratch_types=[
            pltpu.VMEM((R,), jnp.int32),
            pltpu.VMEM((R, DW), jnp.int32),
            pltpu.VMEM((C, D), jnp.float32),
            pltpu.SemaphoreType.DMA,
        ],
    )
    def k(feat_hbm, idx_hbm, out_hbm, idx_v, rows_v, out_v, sem):
        wid = lax.axis_index("s") * 2 + lax.axis_index("c")
        tile_node0 = jnp.minimum(wid * PER_TILE, batch - PER_TILE)

        def chunk_body(ci, carry):
            node0 = tile_node0 + ci * C
            row0 = node0 * S
            pltpu.sync_copy(idx_hbm.at[pl.ds(row0, R)], idx_v)
            cps = []
            for g0, gn in GATHER_SPLITS:
                cps.append(pltpu.async_copy(
                    feat_hbm.at[idx_v.at[pl.ds(g0, gn)]],
                    rows_v.at[pl.ds(g0, gn)],
                    sem,
                ))
            for cp in cps:
                cp.wait()

            def node_body(n, carry2):
                base = n * S
                for g in range(DW // L):
                    acc_lo = None
                    acc_hi = None
                    for s_ in range(S):
                        w = rows_v[base + s_, pl.ds(g * L, L)]
                        lo = plsc.bitcast(w << 16, jnp.float32)
                        hi = plsc.bitcast(w & jnp.int32(-65536),
                                          jnp.float32)
                        acc_lo = lo if acc_lo is None else acc_lo + lo
                        acc_hi = hi if acc_hi is None else acc_hi + hi
                    out_v[n, pl.ds(g * L, L)] = acc_lo * jnp.float32(0.1)
                    out_v[n, pl.ds(DW + g * L, L)] = (
                        acc_hi * jnp.float32(0.1))
                return carry2

            lax.fori_loop(0, C, node_body, 0)
            pltpu.sync_copy(out_v, out_hbm.at[pl.ds(node0, C)])
            return carry

        lax.fori_loop(0, CHUNKS, chunk_body, 0)

    return k(features_pk, idx_flat)


def kernel(features, nodes, to_neighs):
    b = to_neighs.shape[0]
    u = jax.lax.bitcast_convert_type(features, jnp.uint32)
    half = jnp.uint32(0x8000)
    lo = (u[:, :DW] + half) >> 16
    hi = (u[:, DW:] + half) & jnp.uint32(0xFFFF0000)
    features_pk = jax.lax.bitcast_convert_type(hi | lo, jnp.int32)
    idx = to_neighs.astype(jnp.int32).reshape(-1)
    return _sc_mean(features_pk, idx, b)


# speedup vs baseline: 5.0704x; 1.4678x over previous
import functools

import jax
import jax.numpy as jnp
from jax import lax
from jax.experimental import pallas as pl
from jax.experimental.pallas import tpu as pltpu
from jax.experimental.pallas import tpu_sc as plsc

D = 128
DW = D // 2
S = 10
L = 16
NW = 32
C = 32
R = C * S
CHUNKS = 49
PER_TILE = C * CHUNKS
GATHER_SPLITS = ((0, 128), (128, 128), (256, 64))


def _sc_mean(features_pk, idx_flat, batch):
    mesh = plsc.VectorSubcoreMesh(core_axis_name="c", subcore_axis_name="s")

    @functools.partial(
        pl.kernel,
        mesh=mesh,
        out_type=jax.ShapeDtypeStruct((batch, D), jnp.float32),
        compiler_params=pltpu.CompilerParams(needs_layout_passes=False,
                                             use_tc_tiling_on_sc=False),
        scratch_types=[
            pltpu.VMEM((2 * R,), jnp.int32),
            pltpu.VMEM((2 * R, DW), jnp.int32),
            pltpu.VMEM((2 * C, D), jnp.float32),
            pltpu.SemaphoreType.DMA,
            pltpu.SemaphoreType.DMA,
            pltpu.SemaphoreType.DMA,
        ],
    )
    def k(feat_hbm, idx_hbm, out_hbm, idx_v, rows_v, out_v, isem, gsem, osem):
        wid = lax.axis_index("s") * 2 + lax.axis_index("c")
        tile_node0 = jnp.minimum(wid * PER_TILE, batch - PER_TILE)
        tile_row0 = tile_node0 * S

        def i_start(c, boff):
            pltpu.async_copy(
                idx_hbm.at[pl.ds(tile_row0 + c * R, R)],
                idx_v.at[pl.ds(boff, R)], isem)

        def i_wait():
            pltpu.make_async_copy(
                idx_hbm.at[pl.ds(tile_row0, R)],
                idx_v.at[pl.ds(0, R)], isem).wait()

        def g_start(boff):
            for g0, gn in GATHER_SPLITS:
                pltpu.async_copy(
                    feat_hbm.at[idx_v.at[pl.ds(boff + g0, gn)]],
                    rows_v.at[pl.ds(boff + g0, gn)],
                    gsem,
                )

        def g_wait(boff):
            for g0, gn in GATHER_SPLITS:
                pltpu.make_async_copy(
                    feat_hbm.at[idx_v.at[pl.ds(boff + g0, gn)]],
                    rows_v.at[pl.ds(boff + g0, gn)],
                    gsem,
                ).wait()

        def o_start(c, ooff):
            pltpu.async_copy(
                out_v.at[pl.ds(ooff, C)],
                out_hbm.at[pl.ds(tile_node0 + c * C, C)], osem)

        def o_wait():
            pltpu.make_async_copy(
                out_v.at[pl.ds(0, C)],
                out_hbm.at[pl.ds(tile_node0, C)], osem).wait()

        pltpu.sync_copy(idx_hbm.at[pl.ds(tile_row0, R)],
                        idx_v.at[pl.ds(0, R)])
        g_start(0)
        i_start(1, R)

        def chunk_body(c, carry):
            par = lax.rem(c, 2)
            boff = par * R
            boff_n = R - boff
            ooff = par * C

            g_wait(boff)

            @pl.when(c + 2 < CHUNKS)
            def _():
                i_start(c + 2, boff)

            @pl.when(c + 1 < CHUNKS)
            def _():
                i_wait()
                g_start(boff_n)

            @pl.when(c >= 2)
            def _():
                o_wait()

            def node_body(n, carry2):
                base = boff + n * S
                for g in range(DW // L):
                    acc_lo = None
                    acc_hi = None
                    for s_ in range(S):
                        w = rows_v[base + s_, pl.ds(g * L, L)]
                        lo = plsc.bitcast(w << 16, jnp.float32)
                        hi = plsc.bitcast(w & jnp.int32(-65536),
                                          jnp.float32)
                        acc_lo = lo if acc_lo is None else acc_lo + lo
                        acc_hi = hi if acc_hi is None else acc_hi + hi
                    out_v[ooff + n, pl.ds(g * L, L)] = (
                        acc_lo * jnp.float32(0.1))
                    out_v[ooff + n, pl.ds(DW + g * L, L)] = (
                        acc_hi * jnp.float32(0.1))
                return carry2

            lax.fori_loop(0, C, node_body, 0)
            o_start(c, ooff)
            return carry

        lax.fori_loop(0, CHUNKS, chunk_body, 0)
        o_wait()
        o_wait()

    return k(features_pk, idx_flat)


def kernel(features, nodes, to_neighs):
    b = to_neighs.shape[0]
    u = jax.lax.bitcast_convert_type(features, jnp.uint32)
    half = jnp.uint32(0x8000)
    lo = (u[:, :DW] + half) >> 16
    hi = (u[:, DW:] + half) & jnp.uint32(0xFFFF0000)
    features_pk = jax.lax.bitcast_convert_type(hi | lo, jnp.int32)
    idx = to_neighs.astype(jnp.int32).reshape(-1)
    return _sc_mean(features_pk, idx, b)
